# baseline (speedup 1.0000x reference)
import jax
import jax.numpy as jnp
from jax.experimental import pallas as pl
from jax.experimental.pallas import tpu as pltpu

_E = 5
_K = 3


def _fused_moe_kernel(x_ref, wg_ref, bg_ref, we_ref, be_ref, o_ref):
    x = x_ref[...]
    logits = jnp.dot(x, wg_ref[...],
                     preferred_element_type=jnp.float32) + bg_ref[...]
    g = jax.nn.softmax(logits, axis=-1)
    cols = [g[:, e:e + 1] for e in range(_E)]
    ps = []
    for e in range(_E):
        cnt = jnp.zeros_like(cols[e])
        for e2 in range(_E):
            if e2 == e:
                continue
            if e2 < e:
                beats = cols[e2] >= cols[e]
            else:
                beats = cols[e2] > cols[e]
            cnt = cnt + beats.astype(jnp.float32)
        ps.append(jnp.where(cnt < float(_K), cols[e], 0.0))
    p = jnp.concatenate(ps, axis=1)
    acc = jnp.dot(p, be_ref[...], preferred_element_type=jnp.float32)
    for e in range(_E):
        y_e = jnp.dot(x, we_ref[e], preferred_element_type=jnp.float32)
        acc = acc + ps[e] * y_e
    o_ref[...] = acc


def kernel(x, Wg, bg, We, be):
    B, D = x.shape
    E, _, H = We.shape
    Tb = 512
    Hb = 512
    bg2 = bg.reshape(1, E)
    grid = (H // Hb, B // Tb)
    return pl.pallas_call(
        _fused_moe_kernel,
        grid=grid,
        in_specs=[
            pl.BlockSpec((Tb, D), lambda h, t: (t, 0)),
            pl.BlockSpec((D, E), lambda h, t: (0, 0)),
            pl.BlockSpec((1, E), lambda h, t: (0, 0)),
            pl.BlockSpec((E, D, Hb), lambda h, t: (0, 0, h)),
            pl.BlockSpec((E, Hb), lambda h, t: (0, h)),
        ],
        out_specs=pl.BlockSpec((Tb, Hb), lambda h, t: (t, h)),
        out_shape=jax.ShapeDtypeStruct((B, H), jnp.float32),
        compiler_params=pltpu.CompilerParams(vmem_limit_bytes=67108864),
    )(x, Wg, bg2, We, be)


# speedup vs baseline: 1.1316x; 1.1316x over previous
import jax
import jax.numpy as jnp
from jax.experimental import pallas as pl
from jax.experimental.pallas import tpu as pltpu

_E = 5
_K = 3


def _fused_moe_kernel(x_ref, wg_ref, bg_ref, we_ref, be_ref, o_ref):
    x = x_ref[...]
    logits = jnp.dot(x, wg_ref[...],
                     preferred_element_type=jnp.float32) + bg_ref[...]
    g = jax.nn.softmax(logits, axis=-1)
    cols = [g[:, e:e + 1] for e in range(_E)]
    ps = []
    for e in range(_E):
        cnt = jnp.zeros_like(cols[e])
        for e2 in range(_E):
            if e2 == e:
                continue
            if e2 < e:
                beats = cols[e2] >= cols[e]
            else:
                beats = cols[e2] > cols[e]
            cnt = cnt + beats.astype(jnp.float32)
        ps.append(jnp.where(cnt < float(_K), cols[e], 0.0))
    acc = jnp.zeros((x.shape[0], o_ref.shape[1]), jnp.float32)
    for e in range(_E):
        y_e = jnp.dot(x, we_ref[e], preferred_element_type=jnp.float32)
        acc = acc + ps[e] * (y_e + be_ref[e][None, :])
    o_ref[...] = acc


def kernel(x, Wg, bg, We, be):
    B, D = x.shape
    E, _, H = We.shape
    Tb = 512
    Hb = 512
    bg2 = bg.reshape(1, E)
    grid = (H // Hb, B // Tb)
    return pl.pallas_call(
        _fused_moe_kernel,
        grid=grid,
        in_specs=[
            pl.BlockSpec((Tb, D), lambda h, t: (t, 0)),
            pl.BlockSpec((D, E), lambda h, t: (0, 0)),
            pl.BlockSpec((1, E), lambda h, t: (0, 0)),
            pl.BlockSpec((E, D, Hb), lambda h, t: (0, 0, h)),
            pl.BlockSpec((E, Hb), lambda h, t: (0, h)),
        ],
        out_specs=pl.BlockSpec((Tb, Hb), lambda h, t: (t, h)),
        out_shape=jax.ShapeDtypeStruct((B, H), jnp.float32),
        compiler_params=pltpu.CompilerParams(vmem_limit_bytes=67108864),
    )(x, Wg, bg2, We, be)
